# speedup vs baseline: 25.2070x; 25.2070x over previous
import functools

import jax
import jax.numpy as jnp
from jax import lax
from jax.experimental import pallas as pl
from jax.experimental.pallas import tpu as pltpu

N = 10000
E = 320000
DIMS = 16
IN = 8
OUT = 8
NB = 1000


def _front_body(x_ref, w2_ref, a1_ref, a2_ref, h2_ref, s1_ref, s2_ref):
    xb = x_ref[...]
    h2 = jnp.dot(xb, w2_ref[...], preferred_element_type=jnp.float32)
    h2_ref[...] = h2
    s1_ref[...] = jnp.dot(h2, a1_ref[...], preferred_element_type=jnp.float32)
    s2_ref[...] = jnp.dot(h2, a2_ref[...], preferred_element_type=jnp.float32)


def _front(x, w2, a1, a2):
    grid = (N // NB,)
    return pl.pallas_call(
        _front_body,
        grid=grid,
        in_specs=[
            pl.BlockSpec((NB, IN * DIMS), lambda n: (n, 0)),
            pl.BlockSpec((IN * DIMS, OUT * DIMS), lambda n: (0, 0)),
            pl.BlockSpec((OUT * DIMS, DIMS), lambda n: (0, 0)),
            pl.BlockSpec((OUT * DIMS, DIMS), lambda n: (0, 0)),
        ],
        out_specs=[
            pl.BlockSpec((NB, OUT * DIMS), lambda n: (n, 0)),
            pl.BlockSpec((NB, DIMS), lambda n: (n, 0)),
            pl.BlockSpec((NB, DIMS), lambda n: (n, 0)),
        ],
        out_shape=[
            jax.ShapeDtypeStruct((N, OUT * DIMS), jnp.float32),
            jax.ShapeDtypeStruct((N, DIMS), jnp.float32),
            jax.ShapeDtypeStruct((N, DIMS), jnp.float32),
        ],
    )(x, w2, a1, a2)


def _final_body(o_ref, p_ref, out_ref):
    out_ref[...] = jnp.dot(o_ref[...], p_ref[...],
                           preferred_element_type=jnp.float32)


def _final(out2, perm):
    return pl.pallas_call(
        _final_body,
        grid=(N // NB,),
        in_specs=[
            pl.BlockSpec((NB, OUT * DIMS), lambda n: (n, 0)),
            pl.BlockSpec((OUT * DIMS, OUT * DIMS), lambda n: (0, 0)),
        ],
        out_specs=pl.BlockSpec((NB, OUT * DIMS), lambda n: (n, 0)),
        out_shape=jax.ShapeDtypeStruct((N, OUT * DIMS), jnp.float32),
    )(out2, perm)


def kernel(x, edge_index, edge_attr, W, att):
    src = edge_index[0]
    dst = edge_index[1]

    ii = jnp.arange(DIMS)
    kk = jnp.arange(IN)
    oo = jnp.arange(OUT)
    rows = kk[None, :, None] * DIMS + ii[:, None, None]
    cols = oo[None, None, :] * DIMS + ii[:, None, None]
    w2 = jnp.zeros((IN * DIMS, OUT * DIMS), jnp.float32).at[rows, cols].set(
        jnp.transpose(W, (0, 1, 2)))
    r2 = oo[None, :] * DIMS + ii[:, None]
    c2 = jnp.broadcast_to(ii[:, None], (DIMS, OUT))
    a1 = jnp.zeros((OUT * DIMS, DIMS), jnp.float32).at[r2, c2].set(att[:, :OUT])
    a2 = jnp.zeros((OUT * DIMS, DIMS), jnp.float32).at[r2, c2].set(att[:, OUT:2 * OUT])
    w16 = att[:, 2 * OUT]
    pr = oo[None, :] * DIMS + ii[:, None]
    pc = ii[:, None] * OUT + oo[None, :]
    perm = jnp.zeros((OUT * DIMS, OUT * DIMS), jnp.float32).at[pr, pc].set(1.0)

    h2, s1, s2 = _front(x, w2, a1, a2)

    logits = s1[src] + s2[dst] + edge_attr * w16[None, :]
    logits = jnp.where(logits >= 0, logits, 0.2 * logits)
    ex = jnp.exp(logits)
    denom = jax.ops.segment_sum(ex, dst, num_segments=N)
    alpha = ex / (denom[dst] + 1e-16)
    msg = alpha[:, None, :] * h2[src].reshape(E, OUT, DIMS)
    out2 = jax.ops.segment_sum(msg.reshape(E, OUT * DIMS), dst, num_segments=N)

    out = _final(out2, perm)
    return out, alpha, edge_index


# speedup vs baseline: 113.1059x; 4.4871x over previous
import functools

import jax
import jax.numpy as jnp
from jax import lax
from jax.experimental import pallas as pl
from jax.experimental.pallas import tpu as pltpu
from jax.experimental.pallas import tpu_sc as plsc

N = 10000
E = 320000
DIMS = 16
IN = 8
OUT = 8
NB = 1000

NC = 2
NS = 16
NW = NC * NS
EW = E // NW
NPAD = 10240
ZR = NPAD // NS

C1 = 1000
C2 = 200


def _front_body(x_ref, w2_ref, a1_ref, a2_ref, h2_ref, s1_ref, s2_ref):
    xb = x_ref[...]
    h2 = jnp.dot(xb, w2_ref[...], preferred_element_type=jnp.float32)
    h2_ref[...] = h2
    s1_ref[...] = jnp.dot(h2, a1_ref[...], preferred_element_type=jnp.float32)
    s2_ref[...] = jnp.dot(h2, a2_ref[...], preferred_element_type=jnp.float32)


def _front(x, w2, a1, a2):
    return pl.pallas_call(
        _front_body,
        grid=(N // NB,),
        in_specs=[
            pl.BlockSpec((NB, IN * DIMS), lambda n: (n, 0)),
            pl.BlockSpec((IN * DIMS, OUT * DIMS), lambda n: (0, 0)),
            pl.BlockSpec((OUT * DIMS, DIMS), lambda n: (0, 0)),
            pl.BlockSpec((OUT * DIMS, DIMS), lambda n: (0, 0)),
        ],
        out_specs=[
            pl.BlockSpec((NB, OUT * DIMS), lambda n: (n, 0)),
            pl.BlockSpec((NB, DIMS), lambda n: (n, 0)),
            pl.BlockSpec((NB, DIMS), lambda n: (n, 0)),
        ],
        out_shape=[
            jax.ShapeDtypeStruct((N, OUT * DIMS), jnp.float32),
            jax.ShapeDtypeStruct((N, DIMS), jnp.float32),
            jax.ShapeDtypeStruct((N, DIMS), jnp.float32),
        ],
    )(x, w2, a1, a2)


def _mid_body(d_ref, inv_ref):
    d = d_ref[0] + d_ref[1]
    inv_ref[...] = 1.0 / (d + 1e-16)


def _mid(dpart):
    return pl.pallas_call(
        _mid_body,
        grid=(N // NB,),
        in_specs=[pl.BlockSpec((NC, NB, DIMS), lambda n: (0, n, 0))],
        out_specs=pl.BlockSpec((NB, DIMS), lambda n: (n, 0)),
        out_shape=jax.ShapeDtypeStruct((N, DIMS), jnp.float32),
    )(dpart)


def _final_body(o_ref, p_ref, out_ref):
    out_ref[...] = jnp.dot(o_ref[0] + o_ref[1], p_ref[...],
                           preferred_element_type=jnp.float32)


def _final(opart, perm):
    return pl.pallas_call(
        _final_body,
        grid=(N // NB,),
        in_specs=[
            pl.BlockSpec((NC, NB, OUT * DIMS), lambda n: (0, n, 0)),
            pl.BlockSpec((OUT * DIMS, OUT * DIMS), lambda n: (0, 0)),
        ],
        out_specs=pl.BlockSpec((NB, OUT * DIMS), lambda n: (n, 0)),
        out_shape=jax.ShapeDtypeStruct((N, OUT * DIMS), jnp.float32),
    )(opart, perm)


def _pass1_body(src_hbm, dst_hbm, ea_hbm, s1_hbm, s2_hbm, w16_hbm, z16_hbm,
                ex_hbm, dpart_hbm,
                sidx, didx, eabuf, s1buf, s2buf, w16v, denom_sh, sem):
    cid = lax.axis_index("c")
    sid = lax.axis_index("s")
    wid = sid * NC + cid

    pltpu.sync_copy(w16_hbm, w16v)
    pltpu.sync_copy(z16_hbm.at[pl.ds(sid * ZR, ZR)],
                    denom_sh.at[pl.ds(sid * ZR, ZR)])
    plsc.subcore_barrier()

    wv = w16v[...]
    base = wid * EW

    def chunk_body(k, carry):
        off = base + k * C1
        pltpu.sync_copy(src_hbm.at[pl.ds(off, C1)], sidx)
        pltpu.sync_copy(dst_hbm.at[pl.ds(off, C1)], didx)
        pltpu.sync_copy(ea_hbm.at[pl.ds(off, C1)], eabuf)
        pltpu.async_copy(s1_hbm.at[sidx], s1buf, sem).wait()
        pltpu.async_copy(s2_hbm.at[didx], s2buf, sem).wait()

        def edge_body(c, carry2):
            v = s1buf[c] + s2buf[c] + eabuf[c] * wv
            v = jnp.where(v >= 0.0, v, 0.2 * v)
            s1buf[c] = jnp.exp(v)
            return carry2

        lax.fori_loop(0, C1, edge_body, 0, unroll=4)
        pltpu.sync_copy(s1buf, ex_hbm.at[pl.ds(off, C1)])
        pltpu.sync_copy(s1buf, denom_sh.at[didx], add=True)
        return carry

    lax.fori_loop(0, EW // C1, chunk_body, 0)
    plsc.subcore_barrier()
    pltpu.sync_copy(denom_sh.at[pl.ds(sid * ZR, ZR)],
                    dpart_hbm.at[cid, pl.ds(sid * ZR, ZR)])


def _pass1(src, dst, ea, s1, s2, w16, z16):
    mesh = plsc.VectorSubcoreMesh(core_axis_name="c", subcore_axis_name="s")
    f = functools.partial(
        pl.kernel,
        out_type=[
            jax.ShapeDtypeStruct((E, DIMS), jnp.float32),
            jax.ShapeDtypeStruct((NC, NPAD, DIMS), jnp.float32),
        ],
        mesh=mesh,
        compiler_params=pltpu.CompilerParams(use_tc_tiling_on_sc=False),
        scratch_types=[
            pltpu.VMEM((C1,), jnp.int32),
            pltpu.VMEM((C1,), jnp.int32),
            pltpu.VMEM((C1, DIMS), jnp.float32),
            pltpu.VMEM((C1, DIMS), jnp.float32),
            pltpu.VMEM((C1, DIMS), jnp.float32),
            pltpu.VMEM((DIMS,), jnp.float32),
            pltpu.VMEM_SHARED((NPAD, DIMS), jnp.float32),
            pltpu.SemaphoreType.DMA,
        ],
    )(_pass1_body)
    return f(src, dst, ea, s1, s2, w16, z16)


def _pass2_body(src_hbm, dst_hbm, ex_hbm, inv_hbm, h2_hbm, z128_hbm,
                alpha_hbm, opart_hbm,
                sidx, didx, exbuf, invbuf, albuf, hbuf, out_sh, sem):
    cid = lax.axis_index("c")
    sid = lax.axis_index("s")
    wid = sid * NC + cid

    pltpu.sync_copy(z128_hbm.at[pl.ds(sid * ZR, ZR)],
                    out_sh.at[pl.ds(sid * ZR, ZR)])
    plsc.subcore_barrier()

    base = wid * EW

    def chunk_body(k, carry):
        off = base + k * C2
        pltpu.sync_copy(src_hbm.at[pl.ds(off, C2)], sidx)
        pltpu.sync_copy(dst_hbm.at[pl.ds(off, C2)], didx)
        pltpu.sync_copy(ex_hbm.at[pl.ds(off, C2)], exbuf)
        pltpu.async_copy(inv_hbm.at[didx], invbuf, sem).wait()
        pltpu.async_copy(h2_hbm.at[sidx], hbuf, sem).wait()

        def edge_body(c, carry2):
            a = exbuf[c] * invbuf[c]
            albuf[c] = a
            for o in range(OUT):
                hbuf[c, pl.ds(o * DIMS, DIMS)] = (
                    a * hbuf[c, pl.ds(o * DIMS, DIMS)])
            return carry2

        lax.fori_loop(0, C2, edge_body, 0, unroll=2)
        pltpu.sync_copy(albuf, alpha_hbm.at[pl.ds(off, C2)])
        pltpu.sync_copy(hbuf, out_sh.at[didx], add=True)
        return carry

    lax.fori_loop(0, EW // C2, chunk_body, 0)
    plsc.subcore_barrier()
    pltpu.sync_copy(out_sh.at[pl.ds(sid * ZR, ZR)],
                    opart_hbm.at[cid, pl.ds(sid * ZR, ZR)])


def _pass2(src, dst, ex, inv, h2, z128):
    mesh = plsc.VectorSubcoreMesh(core_axis_name="c", subcore_axis_name="s")
    f = functools.partial(
        pl.kernel,
        out_type=[
            jax.ShapeDtypeStruct((E, DIMS), jnp.float32),
            jax.ShapeDtypeStruct((NC, NPAD, OUT * DIMS), jnp.float32),
        ],
        mesh=mesh,
        compiler_params=pltpu.CompilerParams(use_tc_tiling_on_sc=False),
        scratch_types=[
            pltpu.VMEM((C2,), jnp.int32),
            pltpu.VMEM((C2,), jnp.int32),
            pltpu.VMEM((C2, DIMS), jnp.float32),
            pltpu.VMEM((C2, DIMS), jnp.float32),
            pltpu.VMEM((C2, DIMS), jnp.float32),
            pltpu.VMEM((C2, OUT * DIMS), jnp.float32),
            pltpu.VMEM_SHARED((NPAD, OUT * DIMS), jnp.float32),
            pltpu.SemaphoreType.DMA,
        ],
    )(_pass2_body)
    return f(src, dst, ex, inv, h2, z128)


def kernel(x, edge_index, edge_attr, W, att):
    src = edge_index[0]
    dst = edge_index[1]

    ii = jnp.arange(DIMS)
    kk = jnp.arange(IN)
    oo = jnp.arange(OUT)
    rows = kk[None, :, None] * DIMS + ii[:, None, None]
    cols = oo[None, None, :] * DIMS + ii[:, None, None]
    w2 = jnp.zeros((IN * DIMS, OUT * DIMS), jnp.float32).at[rows, cols].set(W)
    r2 = oo[None, :] * DIMS + ii[:, None]
    c2 = jnp.broadcast_to(ii[:, None], (DIMS, OUT))
    a1 = jnp.zeros((OUT * DIMS, DIMS), jnp.float32).at[r2, c2].set(att[:, :OUT])
    a2 = jnp.zeros((OUT * DIMS, DIMS), jnp.float32).at[r2, c2].set(att[:, OUT:2 * OUT])
    w16 = att[:, 2 * OUT]
    pr = oo[None, :] * DIMS + ii[:, None]
    pc = ii[:, None] * OUT + oo[None, :]
    perm = jnp.zeros((OUT * DIMS, OUT * DIMS), jnp.float32).at[pr, pc].set(1.0)
    z16 = jnp.zeros((NPAD, DIMS), jnp.float32)
    z128 = jnp.zeros((NPAD, OUT * DIMS), jnp.float32)

    h2, s1, s2 = _front(x, w2, a1, a2)
    ex, dpart = _pass1(src, dst, edge_attr, s1, s2, w16, z16)
    inv = _mid(dpart)
    alpha, opart = _pass2(src, dst, ex, inv, h2, z128)
    out = _final(opart, perm)
    return out, alpha, edge_index


# speedup vs baseline: 148.3515x; 1.3116x over previous
import functools

import jax
import jax.numpy as jnp
from jax import lax
from jax.experimental import pallas as pl
from jax.experimental.pallas import tpu as pltpu
from jax.experimental.pallas import tpu_sc as plsc

N = 10000
E = 320000
DIMS = 16
IN = 8
OUT = 8
NB = 1000

NC = 2
NS = 16
NW = NC * NS
EW = E // NW
NPAD = 10240
ZR = NPAD // NS

C1 = 200
CH1 = EW // C1
C2 = 40
CH2 = EW // C2

_SC_PARAMS = pltpu.CompilerParams(use_tc_tiling_on_sc=False)


def _front_body(x_ref, w2_ref, a1_ref, a2_ref, h2_ref, s1_ref, s2_ref):
    xb = x_ref[...]
    h2 = jnp.dot(xb, w2_ref[...], preferred_element_type=jnp.float32)
    h2_ref[...] = h2
    s1_ref[...] = jnp.dot(h2, a1_ref[...], preferred_element_type=jnp.float32)
    s2_ref[...] = jnp.dot(h2, a2_ref[...], preferred_element_type=jnp.float32)


def _front(x, w2, a1, a2):
    return pl.pallas_call(
        _front_body,
        grid=(N // NB,),
        in_specs=[
            pl.BlockSpec((NB, IN * DIMS), lambda n: (n, 0)),
            pl.BlockSpec((IN * DIMS, OUT * DIMS), lambda n: (0, 0)),
            pl.BlockSpec((OUT * DIMS, DIMS), lambda n: (0, 0)),
            pl.BlockSpec((OUT * DIMS, DIMS), lambda n: (0, 0)),
        ],
        out_specs=[
            pl.BlockSpec((NB, OUT * DIMS), lambda n: (n, 0)),
            pl.BlockSpec((NB, DIMS), lambda n: (n, 0)),
            pl.BlockSpec((NB, DIMS), lambda n: (n, 0)),
        ],
        out_shape=[
            jax.ShapeDtypeStruct((N, OUT * DIMS), jnp.float32),
            jax.ShapeDtypeStruct((N, DIMS), jnp.float32),
            jax.ShapeDtypeStruct((N, DIMS), jnp.float32),
        ],
    )(x, w2, a1, a2)


def _mid_body(d_ref, inv_ref):
    d = d_ref[0] + d_ref[1]
    inv_ref[...] = 1.0 / (d + 1e-16)


def _mid(dpart):
    return pl.pallas_call(
        _mid_body,
        grid=(NPAD // NB,),
        in_specs=[pl.BlockSpec((NC, NB, DIMS), lambda n: (0, n, 0))],
        out_specs=pl.BlockSpec((NB, DIMS), lambda n: (n, 0)),
        out_shape=jax.ShapeDtypeStruct((NPAD, DIMS), jnp.float32),
    )(dpart)


def _final_body(o_ref, p_ref, out_ref):
    out_ref[...] = jnp.dot(o_ref[0] + o_ref[1], p_ref[...],
                           preferred_element_type=jnp.float32)


def _final(opart, perm):
    return pl.pallas_call(
        _final_body,
        grid=(N // NB,),
        in_specs=[
            pl.BlockSpec((NC, NB, OUT * DIMS), lambda n: (0, n, 0)),
            pl.BlockSpec((OUT * DIMS, OUT * DIMS), lambda n: (0, 0)),
        ],
        out_specs=pl.BlockSpec((NB, OUT * DIMS), lambda n: (n, 0)),
        out_shape=jax.ShapeDtypeStruct((N, OUT * DIMS), jnp.float32),
    )(opart, perm)


def _pass1_body(src3_hbm, dst3_hbm, ea_hbm, s1_hbm, s2_hbm, w16_hbm, z16_hbm,
                ex_hbm, dpart_hbm,
                sidx, didx, eab0, eab1, s1b0, s1b1, s2b0, s2b1, exo0, exo1,
                w16v, denom_sh, sem0, sem1, xsem0, xsem1):
    cid = lax.axis_index("c")
    sid = lax.axis_index("s")
    wid = sid * NC + cid
    base = wid * EW

    eab = [eab0, eab1]
    s1b = [s1b0, s1b1]
    s2b = [s2b0, s2b1]
    exo = [exo0, exo1]
    sems = [sem0, sem1]
    xsems = [xsem0, xsem1]

    pltpu.sync_copy(w16_hbm, w16v)
    pltpu.sync_copy(src3_hbm.at[wid], sidx)
    pltpu.sync_copy(dst3_hbm.at[wid], didx)
    pltpu.sync_copy(z16_hbm.at[pl.ds(sid * ZR, ZR)],
                    denom_sh.at[pl.ds(sid * ZR, ZR)])
    plsc.subcore_barrier()

    wv = w16v[...]

    def issue(k, slot):
        off = base + k * C1
        pltpu.async_copy(ea_hbm.at[pl.ds(off, C1)], eab[slot], sems[slot])
        pltpu.async_copy(s1_hbm.at[sidx.at[k]], s1b[slot], sems[slot])
        pltpu.async_copy(s2_hbm.at[didx.at[k]], s2b[slot], sems[slot])

    def step(k, slot):
        off = base + k * C1

        @pl.when(k >= 2)
        def _():
            pltpu.make_async_copy(
                exo[slot], ex_hbm.at[pl.ds(base + (k - 2) * C1, C1)],
                xsems[slot]).wait()

        pltpu.make_async_copy(ea_hbm.at[pl.ds(off, C1)], eab[slot],
                              sems[slot]).wait()
        pltpu.make_async_copy(s1_hbm.at[sidx.at[k]], s1b[slot],
                              sems[slot]).wait()
        pltpu.make_async_copy(s2_hbm.at[didx.at[k]], s2b[slot],
                              sems[slot]).wait()

        def edge_body(c, carry):
            v = s1b[slot][c] + s2b[slot][c] + eab[slot][c] * wv
            v = jnp.where(v >= 0.0, v, 0.2 * v)
            exo[slot][c] = jnp.exp(v)
            return carry

        lax.fori_loop(0, C1, edge_body, 0, unroll=4)

        pltpu.sync_copy(exo[slot], denom_sh.at[didx.at[k]], add=True)
        pltpu.async_copy(exo[slot], ex_hbm.at[pl.ds(off, C1)], xsems[slot])

        @pl.when(k + 2 < CH1)
        def _():
            issue(k + 2, slot)

    issue(0, 0)
    issue(1, 1)

    def pair(j, carry):
        step(2 * j, 0)
        step(2 * j + 1, 1)
        return carry

    lax.fori_loop(0, CH1 // 2, pair, 0)

    pltpu.make_async_copy(exo0, ex_hbm.at[pl.ds(base + (CH1 - 2) * C1, C1)],
                          xsem0).wait()
    pltpu.make_async_copy(exo1, ex_hbm.at[pl.ds(base + (CH1 - 1) * C1, C1)],
                          xsem1).wait()

    plsc.subcore_barrier()
    pltpu.sync_copy(denom_sh.at[pl.ds(sid * ZR, ZR)],
                    dpart_hbm.at[cid, pl.ds(sid * ZR, ZR)])


def _pass1(src3, dst3, ea, s1, s2, w16, z16):
    mesh = plsc.VectorSubcoreMesh(core_axis_name="c", subcore_axis_name="s")
    f = functools.partial(
        pl.kernel,
        out_type=[
            jax.ShapeDtypeStruct((E, DIMS), jnp.float32),
            jax.ShapeDtypeStruct((NC, NPAD, DIMS), jnp.float32),
        ],
        mesh=mesh,
        compiler_params=_SC_PARAMS,
        scratch_types=[
            pltpu.VMEM((CH1, C1), jnp.int32),
            pltpu.VMEM((CH1, C1), jnp.int32),
            pltpu.VMEM((C1, DIMS), jnp.float32),
            pltpu.VMEM((C1, DIMS), jnp.float32),
            pltpu.VMEM((C1, DIMS), jnp.float32),
            pltpu.VMEM((C1, DIMS), jnp.float32),
            pltpu.VMEM((C1, DIMS), jnp.float32),
            pltpu.VMEM((C1, DIMS), jnp.float32),
            pltpu.VMEM((C1, DIMS), jnp.float32),
            pltpu.VMEM((C1, DIMS), jnp.float32),
            pltpu.VMEM((DIMS,), jnp.float32),
            pltpu.VMEM_SHARED((NPAD, DIMS), jnp.float32),
            pltpu.SemaphoreType.DMA,
            pltpu.SemaphoreType.DMA,
            pltpu.SemaphoreType.DMA,
            pltpu.SemaphoreType.DMA,
        ],
    )(_pass1_body)
    return f(src3, dst3, ea, s1, s2, w16, z16)


def _pass2_body(src3_hbm, dst3_hbm, ex_hbm, inv_hbm, h2_hbm, z128_hbm,
                alpha_hbm, opart_hbm,
                sidx, didx, exb0, exb1, ivb0, ivb1, hb0, hb1, alb0, alb1,
                out_sh, sem0, sem1, asem0, asem1):
    cid = lax.axis_index("c")
    sid = lax.axis_index("s")
    wid = sid * NC + cid
    base = wid * EW

    exb = [exb0, exb1]
    ivb = [ivb0, ivb1]
    hb = [hb0, hb1]
    alb = [alb0, alb1]
    sems = [sem0, sem1]
    asems = [asem0, asem1]

    pltpu.sync_copy(src3_hbm.at[wid], sidx)
    pltpu.sync_copy(dst3_hbm.at[wid], didx)
    pltpu.sync_copy(z128_hbm.at[pl.ds(sid * ZR, ZR)],
                    out_sh.at[pl.ds(sid * ZR, ZR)])
    plsc.subcore_barrier()

    def issue(k, slot):
        off = base + k * C2
        pltpu.async_copy(ex_hbm.at[pl.ds(off, C2)], exb[slot], sems[slot])
        pltpu.async_copy(inv_hbm.at[didx.at[k]], ivb[slot], sems[slot])
        pltpu.async_copy(h2_hbm.at[sidx.at[k]], hb[slot], sems[slot])

    def step(k, slot):
        off = base + k * C2

        @pl.when(k >= 2)
        def _():
            pltpu.make_async_copy(
                alb[slot], alpha_hbm.at[pl.ds(base + (k - 2) * C2, C2)],
                asems[slot]).wait()

        pltpu.make_async_copy(ex_hbm.at[pl.ds(off, C2)], exb[slot],
                              sems[slot]).wait()
        pltpu.make_async_copy(inv_hbm.at[didx.at[k]], ivb[slot],
                              sems[slot]).wait()
        pltpu.make_async_copy(h2_hbm.at[sidx.at[k]], hb[slot],
                              sems[slot]).wait()

        def edge_body(c, carry):
            a = exb[slot][c] * ivb[slot][c]
            alb[slot][c] = a
            for o in range(OUT):
                hb[slot][c, pl.ds(o * DIMS, DIMS)] = (
                    a * hb[slot][c, pl.ds(o * DIMS, DIMS)])
            return carry

        lax.fori_loop(0, C2, edge_body, 0, unroll=2)

        pltpu.sync_copy(hb[slot], out_sh.at[didx.at[k]], add=True)
        pltpu.async_copy(alb[slot], alpha_hbm.at[pl.ds(off, C2)], asems[slot])

        @pl.when(k + 2 < CH2)
        def _():
            issue(k + 2, slot)

    issue(0, 0)
    issue(1, 1)

    def pair(j, carry):
        step(2 * j, 0)
        step(2 * j + 1, 1)
        return carry

    lax.fori_loop(0, CH2 // 2, pair, 0)

    pltpu.make_async_copy(alb0, alpha_hbm.at[pl.ds(base + (CH2 - 2) * C2, C2)],
                          asem0).wait()
    pltpu.make_async_copy(alb1, alpha_hbm.at[pl.ds(base + (CH2 - 1) * C2, C2)],
                          asem1).wait()

    plsc.subcore_barrier()
    pltpu.sync_copy(out_sh.at[pl.ds(sid * ZR, ZR)],
                    opart_hbm.at[cid, pl.ds(sid * ZR, ZR)])


def _pass2(src3, dst3, ex, inv, h2, z128):
    mesh = plsc.VectorSubcoreMesh(core_axis_name="c", subcore_axis_name="s")
    f = functools.partial(
        pl.kernel,
        out_type=[
            jax.ShapeDtypeStruct((E, DIMS), jnp.float32),
            jax.ShapeDtypeStruct((NC, NPAD, OUT * DIMS), jnp.float32),
        ],
        mesh=mesh,
        compiler_params=_SC_PARAMS,
        scratch_types=[
            pltpu.VMEM((CH2, C2), jnp.int32),
            pltpu.VMEM((CH2, C2), jnp.int32),
            pltpu.VMEM((C2, DIMS), jnp.float32),
            pltpu.VMEM((C2, DIMS), jnp.float32),
            pltpu.VMEM((C2, DIMS), jnp.float32),
            pltpu.VMEM((C2, DIMS), jnp.float32),
            pltpu.VMEM((C2, OUT * DIMS), jnp.float32),
            pltpu.VMEM((C2, OUT * DIMS), jnp.float32),
            pltpu.VMEM((C2, DIMS), jnp.float32),
            pltpu.VMEM((C2, DIMS), jnp.float32),
            pltpu.VMEM_SHARED((NPAD, OUT * DIMS), jnp.float32),
            pltpu.SemaphoreType.DMA,
            pltpu.SemaphoreType.DMA,
            pltpu.SemaphoreType.DMA,
            pltpu.SemaphoreType.DMA,
        ],
    )(_pass2_body)
    return f(src3, dst3, ex, inv, h2, z128)


def kernel(x, edge_index, edge_attr, W, att):
    src = edge_index[0]
    dst = edge_index[1]

    ii = jnp.arange(DIMS)
    kk = jnp.arange(IN)
    oo = jnp.arange(OUT)
    rows = kk[None, :, None] * DIMS + ii[:, None, None]
    cols = oo[None, None, :] * DIMS + ii[:, None, None]
    w2 = jnp.zeros((IN * DIMS, OUT * DIMS), jnp.float32).at[rows, cols].set(W)
    r2 = oo[None, :] * DIMS + ii[:, None]
    c2 = jnp.broadcast_to(ii[:, None], (DIMS, OUT))
    a1 = jnp.zeros((OUT * DIMS, DIMS), jnp.float32).at[r2, c2].set(att[:, :OUT])
    a2 = jnp.zeros((OUT * DIMS, DIMS), jnp.float32).at[r2, c2].set(att[:, OUT:2 * OUT])
    w16 = att[:, 2 * OUT]
    pr = oo[None, :] * DIMS + ii[:, None]
    pc = ii[:, None] * OUT + oo[None, :]
    perm = jnp.zeros((OUT * DIMS, OUT * DIMS), jnp.float32).at[pr, pc].set(1.0)
    z16 = jnp.zeros((NPAD, DIMS), jnp.float32)
    z128 = jnp.zeros((NPAD, OUT * DIMS), jnp.float32)

    src31 = src.reshape(NW, CH1, C1)
    dst31 = dst.reshape(NW, CH1, C1)
    src32 = src.reshape(NW, CH2, C2)
    dst32 = dst.reshape(NW, CH2, C2)

    h2, s1, s2 = _front(x, w2, a1, a2)
    ex, dpart = _pass1(src31, dst31, edge_attr, s1, s2, w16, z16)
    inv = _mid(dpart)
    alpha, opart = _pass2(src32, dst32, ex, inv, h2, z128)
    out = _final(opart, perm)
    return out, alpha, edge_index


# speedup vs baseline: 152.6992x; 1.0293x over previous
import functools

import jax
import jax.numpy as jnp
from jax import lax
from jax.experimental import pallas as pl
from jax.experimental.pallas import tpu as pltpu
from jax.experimental.pallas import tpu_sc as plsc

N = 10000
E = 320000
DIMS = 16
IN = 8
OUT = 8
NB = 1000

NC = 2
NS = 16
NW = NC * NS
EW = E // NW
NPAD = 10240
ZR = NPAD // NS

C1 = 200
CH1 = EW // C1
C2 = 40
CH2 = EW // C2

_SC_PARAMS = pltpu.CompilerParams(use_tc_tiling_on_sc=False)


def _front_body(x_ref, w2_ref, a1_ref, a2_ref, h2_ref, s1_ref, s2_ref):
    xb = x_ref[...]
    h2 = jnp.dot(xb, w2_ref[...], preferred_element_type=jnp.float32)
    h2_ref[...] = h2
    s1_ref[...] = jnp.dot(h2, a1_ref[...], preferred_element_type=jnp.float32)
    s2_ref[...] = jnp.dot(h2, a2_ref[...], preferred_element_type=jnp.float32)


def _front(x, w2, a1, a2):
    return pl.pallas_call(
        _front_body,
        grid=(N // NB,),
        in_specs=[
            pl.BlockSpec((NB, IN * DIMS), lambda n: (n, 0)),
            pl.BlockSpec((IN * DIMS, OUT * DIMS), lambda n: (0, 0)),
            pl.BlockSpec((OUT * DIMS, DIMS), lambda n: (0, 0)),
            pl.BlockSpec((OUT * DIMS, DIMS), lambda n: (0, 0)),
        ],
        out_specs=[
            pl.BlockSpec((NB, OUT * DIMS), lambda n: (n, 0)),
            pl.BlockSpec((NB, DIMS), lambda n: (n, 0)),
            pl.BlockSpec((NB, DIMS), lambda n: (n, 0)),
        ],
        out_shape=[
            jax.ShapeDtypeStruct((N, OUT * DIMS), jnp.float32),
            jax.ShapeDtypeStruct((N, DIMS), jnp.float32),
            jax.ShapeDtypeStruct((N, DIMS), jnp.float32),
        ],
    )(x, w2, a1, a2)


def _mid_body(d_ref, inv_ref):
    d = d_ref[0] + d_ref[1]
    inv_ref[...] = 1.0 / (d + 1e-16)


def _mid(dpart):
    return pl.pallas_call(
        _mid_body,
        grid=(NPAD // NB,),
        in_specs=[pl.BlockSpec((NC, NB, DIMS), lambda n: (0, n, 0))],
        out_specs=pl.BlockSpec((NB, DIMS), lambda n: (n, 0)),
        out_shape=jax.ShapeDtypeStruct((NPAD, DIMS), jnp.float32),
    )(dpart)


def _final_body(o_ref, p_ref, out_ref):
    out_ref[...] = jnp.dot(o_ref[0] + o_ref[1], p_ref[...],
                           preferred_element_type=jnp.float32)


def _final(opart, perm):
    return pl.pallas_call(
        _final_body,
        grid=(N // NB,),
        in_specs=[
            pl.BlockSpec((NC, NB, OUT * DIMS), lambda n: (0, n, 0)),
            pl.BlockSpec((OUT * DIMS, OUT * DIMS), lambda n: (0, 0)),
        ],
        out_specs=pl.BlockSpec((NB, OUT * DIMS), lambda n: (n, 0)),
        out_shape=jax.ShapeDtypeStruct((N, OUT * DIMS), jnp.float32),
    )(opart, perm)


def _pass1_body(ei_hbm, ea_hbm, s1_hbm, s2_hbm, w16_hbm, z16_hbm,
                ex_hbm, dpart_hbm,
                sidx, didx, eab0, eab1, s1b0, s1b1, s2b0, s2b1, exo0, exo1,
                w16v, denom_sh, sem0, sem1, xsem0, xsem1):
    cid = lax.axis_index("c")
    sid = lax.axis_index("s")
    wid = sid * NC + cid
    base = wid * EW

    eab = [eab0, eab1]
    s1b = [s1b0, s1b1]
    s2b = [s2b0, s2b1]
    exo = [exo0, exo1]
    sems = [sem0, sem1]
    xsems = [xsem0, xsem1]

    pltpu.sync_copy(w16_hbm, w16v)
    pltpu.sync_copy(ei_hbm.at[0, pl.ds(base, EW)], sidx)
    pltpu.sync_copy(ei_hbm.at[1, pl.ds(base, EW)], didx)
    pltpu.sync_copy(z16_hbm.at[pl.ds(sid * ZR, ZR)],
                    denom_sh.at[pl.ds(sid * ZR, ZR)])
    plsc.subcore_barrier()

    wv = w16v[...]

    def issue(k, slot):
        off = base + k * C1
        loc = k * C1
        pltpu.async_copy(ea_hbm.at[pl.ds(off, C1)], eab[slot], sems[slot])
        pltpu.async_copy(s1_hbm.at[sidx.at[pl.ds(loc, C1)]], s1b[slot],
                         sems[slot])
        pltpu.async_copy(s2_hbm.at[didx.at[pl.ds(loc, C1)]], s2b[slot],
                         sems[slot])

    def step(k, slot):
        off = base + k * C1

        @pl.when(k >= 2)
        def _():
            pltpu.make_async_copy(
                exo[slot], ex_hbm.at[pl.ds(base + (k - 2) * C1, C1)],
                xsems[slot]).wait()

        pltpu.make_async_copy(ea_hbm.at[pl.ds(off, C1)], eab[slot],
                              sems[slot]).wait()
        pltpu.make_async_copy(s1_hbm.at[sidx.at[pl.ds(k * C1, C1)]],
                              s1b[slot], sems[slot]).wait()
        pltpu.make_async_copy(s2_hbm.at[didx.at[pl.ds(k * C1, C1)]],
                              s2b[slot], sems[slot]).wait()

        def edge_body(c, carry):
            v = s1b[slot][c] + s2b[slot][c] + eab[slot][c] * wv
            v = jnp.where(v >= 0.0, v, 0.2 * v)
            exo[slot][c] = jnp.exp(v)
            return carry

        lax.fori_loop(0, C1, edge_body, 0, unroll=4)

        pltpu.sync_copy(exo[slot], denom_sh.at[didx.at[pl.ds(k * C1, C1)]],
                        add=True)
        pltpu.async_copy(exo[slot], ex_hbm.at[pl.ds(off, C1)], xsems[slot])

        @pl.when(k + 2 < CH1)
        def _():
            issue(k + 2, slot)

    issue(0, 0)
    issue(1, 1)

    def pair(j, carry):
        step(2 * j, 0)
        step(2 * j + 1, 1)
        return carry

    lax.fori_loop(0, CH1 // 2, pair, 0)

    pltpu.make_async_copy(exo0, ex_hbm.at[pl.ds(base + (CH1 - 2) * C1, C1)],
                          xsem0).wait()
    pltpu.make_async_copy(exo1, ex_hbm.at[pl.ds(base + (CH1 - 1) * C1, C1)],
                          xsem1).wait()

    plsc.subcore_barrier()
    pltpu.sync_copy(denom_sh.at[pl.ds(sid * ZR, ZR)],
                    dpart_hbm.at[cid, pl.ds(sid * ZR, ZR)])


def _pass1(ei, ea, s1, s2, w16, z16):
    mesh = plsc.VectorSubcoreMesh(core_axis_name="c", subcore_axis_name="s")
    f = functools.partial(
        pl.kernel,
        out_type=[
            jax.ShapeDtypeStruct((E, DIMS), jnp.float32),
            jax.ShapeDtypeStruct((NC, NPAD, DIMS), jnp.float32),
        ],
        mesh=mesh,
        compiler_params=_SC_PARAMS,
        scratch_types=[
            pltpu.VMEM((EW,), jnp.int32),
            pltpu.VMEM((EW,), jnp.int32),
            pltpu.VMEM((C1, DIMS), jnp.float32),
            pltpu.VMEM((C1, DIMS), jnp.float32),
            pltpu.VMEM((C1, DIMS), jnp.float32),
            pltpu.VMEM((C1, DIMS), jnp.float32),
            pltpu.VMEM((C1, DIMS), jnp.float32),
            pltpu.VMEM((C1, DIMS), jnp.float32),
            pltpu.VMEM((C1, DIMS), jnp.float32),
            pltpu.VMEM((C1, DIMS), jnp.float32),
            pltpu.VMEM((DIMS,), jnp.float32),
            pltpu.VMEM_SHARED((NPAD, DIMS), jnp.float32),
            pltpu.SemaphoreType.DMA,
            pltpu.SemaphoreType.DMA,
            pltpu.SemaphoreType.DMA,
            pltpu.SemaphoreType.DMA,
        ],
    )(_pass1_body)
    return f(ei, ea, s1, s2, w16, z16)


def _pass2_body(ei_hbm, ex_hbm, inv_hbm, h2_hbm, z128_hbm,
                alpha_hbm, opart_hbm,
                sidx, didx, exb0, exb1, ivb0, ivb1, hb0, hb1, alb0, alb1,
                out_sh, sem0, sem1, asem0, asem1):
    cid = lax.axis_index("c")
    sid = lax.axis_index("s")
    wid = sid * NC + cid
    base = wid * EW

    exb = [exb0, exb1]
    ivb = [ivb0, ivb1]
    hb = [hb0, hb1]
    alb = [alb0, alb1]
    sems = [sem0, sem1]
    asems = [asem0, asem1]

    pltpu.sync_copy(ei_hbm.at[0, pl.ds(base, EW)], sidx)
    pltpu.sync_copy(ei_hbm.at[1, pl.ds(base, EW)], didx)
    pltpu.sync_copy(z128_hbm.at[pl.ds(sid * ZR, ZR)],
                    out_sh.at[pl.ds(sid * ZR, ZR)])
    plsc.subcore_barrier()

    def issue(k, slot):
        off = base + k * C2
        loc = k * C2
        pltpu.async_copy(ex_hbm.at[pl.ds(off, C2)], exb[slot], sems[slot])
        pltpu.async_copy(inv_hbm.at[didx.at[pl.ds(loc, C2)]], ivb[slot],
                         sems[slot])
        pltpu.async_copy(h2_hbm.at[sidx.at[pl.ds(loc, C2)]], hb[slot],
                         sems[slot])

    def step(k, slot):
        off = base + k * C2

        @pl.when(k >= 2)
        def _():
            pltpu.make_async_copy(
                alb[slot], alpha_hbm.at[pl.ds(base + (k - 2) * C2, C2)],
                asems[slot]).wait()

        pltpu.make_async_copy(ex_hbm.at[pl.ds(off, C2)], exb[slot],
                              sems[slot]).wait()
        pltpu.make_async_copy(inv_hbm.at[didx.at[pl.ds(k * C2, C2)]],
                              ivb[slot], sems[slot]).wait()
        pltpu.make_async_copy(h2_hbm.at[sidx.at[pl.ds(k * C2, C2)]],
                              hb[slot], sems[slot]).wait()

        def edge_body(c, carry):
            a = exb[slot][c] * ivb[slot][c]
            alb[slot][c] = a
            for o in range(OUT):
                hb[slot][c, pl.ds(o * DIMS, DIMS)] = (
                    a * hb[slot][c, pl.ds(o * DIMS, DIMS)])
            return carry

        lax.fori_loop(0, C2, edge_body, 0, unroll=2)

        pltpu.sync_copy(hb[slot], out_sh.at[didx.at[pl.ds(k * C2, C2)]],
                        add=True)
        pltpu.async_copy(alb[slot], alpha_hbm.at[pl.ds(off, C2)], asems[slot])

        @pl.when(k + 2 < CH2)
        def _():
            issue(k + 2, slot)

    issue(0, 0)
    issue(1, 1)

    def pair(j, carry):
        step(2 * j, 0)
        step(2 * j + 1, 1)
        return carry

    lax.fori_loop(0, CH2 // 2, pair, 0)

    pltpu.make_async_copy(alb0, alpha_hbm.at[pl.ds(base + (CH2 - 2) * C2, C2)],
                          asem0).wait()
    pltpu.make_async_copy(alb1, alpha_hbm.at[pl.ds(base + (CH2 - 1) * C2, C2)],
                          asem1).wait()

    plsc.subcore_barrier()
    pltpu.sync_copy(out_sh.at[pl.ds(sid * ZR, ZR)],
                    opart_hbm.at[cid, pl.ds(sid * ZR, ZR)])


def _pass2(ei, ex, inv, h2, z128):
    mesh = plsc.VectorSubcoreMesh(core_axis_name="c", subcore_axis_name="s")
    f = functools.partial(
        pl.kernel,
        out_type=[
            jax.ShapeDtypeStruct((E, DIMS), jnp.float32),
            jax.ShapeDtypeStruct((NC, NPAD, OUT * DIMS), jnp.float32),
        ],
        mesh=mesh,
        compiler_params=_SC_PARAMS,
        scratch_types=[
            pltpu.VMEM((EW,), jnp.int32),
            pltpu.VMEM((EW,), jnp.int32),
            pltpu.VMEM((C2, DIMS), jnp.float32),
            pltpu.VMEM((C2, DIMS), jnp.float32),
            pltpu.VMEM((C2, DIMS), jnp.float32),
            pltpu.VMEM((C2, DIMS), jnp.float32),
            pltpu.VMEM((C2, OUT * DIMS), jnp.float32),
            pltpu.VMEM((C2, OUT * DIMS), jnp.float32),
            pltpu.VMEM((C2, DIMS), jnp.float32),
            pltpu.VMEM((C2, DIMS), jnp.float32),
            pltpu.VMEM_SHARED((NPAD, OUT * DIMS), jnp.float32),
            pltpu.SemaphoreType.DMA,
            pltpu.SemaphoreType.DMA,
            pltpu.SemaphoreType.DMA,
            pltpu.SemaphoreType.DMA,
        ],
    )(_pass2_body)
    return f(ei, ex, inv, h2, z128)


def kernel(x, edge_index, edge_attr, W, att):
    r = jnp.arange(OUT * DIMS)
    wp = jnp.transpose(W, (1, 2, 0)).reshape(IN, OUT * DIMS)
    diag = (r[:, None] % DIMS == r[None, :] % DIMS).astype(jnp.float32)
    w2 = jnp.repeat(wp, DIMS, axis=0) * diag
    sel = (r[:, None] % DIMS == jnp.arange(DIMS)[None, :]).astype(jnp.float32)
    a1 = att[:, :OUT].T.reshape(-1)[:, None] * sel
    a2 = att[:, OUT:2 * OUT].T.reshape(-1)[:, None] * sel
    w16 = att[:, 2 * OUT]
    perm = (jnp.arange(OUT * DIMS)[None, :]
            == ((r % DIMS) * OUT + r // DIMS)[:, None]).astype(jnp.float32)
    z16 = jnp.zeros((NPAD, DIMS), jnp.float32)
    z128 = jnp.zeros((NPAD, OUT * DIMS), jnp.float32)

    h2, s1, s2 = _front(x, w2, a1, a2)
    ex, dpart = _pass1(edge_index, edge_attr, s1, s2, w16, z16)
    inv = _mid(dpart)
    alpha, opart = _pass2(edge_index, ex, inv, h2, z128)
    out = _final(opart, perm)
    return out, alpha, edge_index


# speedup vs baseline: 175.6784x; 1.1505x over previous
import functools

import jax
import jax.numpy as jnp
from jax import lax
from jax.experimental import pallas as pl
from jax.experimental.pallas import tpu as pltpu
from jax.experimental.pallas import tpu_sc as plsc

N = 10000
E = 320000
DIMS = 16
IN = 8
OUT = 8
NB = 1000

NC = 2
NS = 16
NW = NC * NS
EW = E // NW
NPAD = 10240
ZR = NPAD // NS

C1 = 200
CH1 = EW // C1
C2 = 40
CH2 = EW // C2

_SC_PARAMS = pltpu.CompilerParams(use_tc_tiling_on_sc=False)


def _front_body(x_ref, w2_ref, a1_ref, a2_ref, h2_ref, s1_ref, s2_ref):
    xb = x_ref[...]
    h2 = jnp.dot(xb, w2_ref[...], preferred_element_type=jnp.float32)
    h2_ref[...] = h2
    s1_ref[...] = jnp.dot(h2, a1_ref[...], preferred_element_type=jnp.float32)
    s2_ref[...] = jnp.dot(h2, a2_ref[...], preferred_element_type=jnp.float32)


def _front(x, w2, a1, a2):
    return pl.pallas_call(
        _front_body,
        grid=(N // NB,),
        in_specs=[
            pl.BlockSpec((NB, IN * DIMS), lambda n: (n, 0)),
            pl.BlockSpec((IN * DIMS, OUT * DIMS), lambda n: (0, 0)),
            pl.BlockSpec((OUT * DIMS, DIMS), lambda n: (0, 0)),
            pl.BlockSpec((OUT * DIMS, DIMS), lambda n: (0, 0)),
        ],
        out_specs=[
            pl.BlockSpec((NB, OUT * DIMS), lambda n: (n, 0)),
            pl.BlockSpec((NB, DIMS), lambda n: (n, 0)),
            pl.BlockSpec((NB, DIMS), lambda n: (n, 0)),
        ],
        out_shape=[
            jax.ShapeDtypeStruct((N, OUT * DIMS), jnp.float32),
            jax.ShapeDtypeStruct((N, DIMS), jnp.float32),
            jax.ShapeDtypeStruct((N, DIMS), jnp.float32),
        ],
    )(x, w2, a1, a2)


def _mid_body(d_ref, inv_ref):
    d = d_ref[0] + d_ref[1]
    inv_ref[...] = 1.0 / (d + 1e-16)


def _mid(dpart):
    return pl.pallas_call(
        _mid_body,
        grid=(NPAD // NB,),
        in_specs=[pl.BlockSpec((NC, NB, DIMS), lambda n: (0, n, 0))],
        out_specs=pl.BlockSpec((NB, DIMS), lambda n: (n, 0)),
        out_shape=jax.ShapeDtypeStruct((NPAD, DIMS), jnp.float32),
    )(dpart)


def _final_body(o_ref, p_ref, out_ref):
    out_ref[...] = jnp.dot(o_ref[0] + o_ref[1], p_ref[...],
                           preferred_element_type=jnp.float32)


def _final(opart, perm):
    return pl.pallas_call(
        _final_body,
        grid=(N // NB,),
        in_specs=[
            pl.BlockSpec((NC, NB, OUT * DIMS), lambda n: (0, n, 0)),
            pl.BlockSpec((OUT * DIMS, OUT * DIMS), lambda n: (0, 0)),
        ],
        out_specs=pl.BlockSpec((NB, OUT * DIMS), lambda n: (n, 0)),
        out_shape=jax.ShapeDtypeStruct((N, OUT * DIMS), jnp.float32),
    )(opart, perm)


def _pass1_body(ei_hbm, ea_hbm, s1_hbm, s2_hbm, w16_hbm, z16_hbm,
                ex_hbm, dpart_hbm,
                sidx, didx, eab0, eab1, s1b0, s1b1, s2b0, s2b1, exo0, exo1,
                exs0, exs1, w16v, denom_sh, sem0, sem1, xsem0, xsem1):
    cid = lax.axis_index("c")
    sid = lax.axis_index("s")
    wid = sid * NC + cid
    base = wid * EW

    eab = [eab0, eab1]
    s1b = [s1b0, s1b1]
    s2b = [s2b0, s2b1]
    exo = [exo0, exo1]
    exs = [exs0, exs1]
    sems = [sem0, sem1]
    xsems = [xsem0, xsem1]

    pltpu.sync_copy(w16_hbm, w16v)
    pltpu.sync_copy(ei_hbm.at[0, pl.ds(base, EW)], sidx)
    pltpu.sync_copy(ei_hbm.at[1, pl.ds(base, EW)], didx)
    pltpu.sync_copy(z16_hbm.at[pl.ds(sid * ZR, ZR)],
                    denom_sh.at[pl.ds(sid * ZR, ZR)])
    plsc.subcore_barrier()

    wv = w16v[...]

    def issue(k, slot):
        off = base + k * C1
        loc = k * C1
        pltpu.async_copy(ea_hbm.at[pl.ds(off, C1)], eab[slot], sems[slot])
        pltpu.async_copy(s1_hbm.at[sidx.at[pl.ds(loc, C1)]], s1b[slot],
                         sems[slot])
        pltpu.async_copy(s2_hbm.at[didx.at[pl.ds(loc, C1)]], s2b[slot],
                         sems[slot])

    def step(k, slot):
        off = base + k * C1
        off8 = (base + k * C1) // 8

        @pl.when(k >= 2)
        def _():
            pltpu.make_async_copy(
                exo[slot], ex_hbm.at[pl.ds((base + (k - 2) * C1) // 8, C1 // 8)],
                xsems[slot]).wait()

        pltpu.make_async_copy(ea_hbm.at[pl.ds(off, C1)], eab[slot],
                              sems[slot]).wait()
        pltpu.make_async_copy(s1_hbm.at[sidx.at[pl.ds(k * C1, C1)]],
                              s1b[slot], sems[slot]).wait()
        pltpu.make_async_copy(s2_hbm.at[didx.at[pl.ds(k * C1, C1)]],
                              s2b[slot], sems[slot]).wait()

        def edge_body(c2, carry):
            for j in range(8):
                c = 8 * c2 + j
                v = s1b[slot][c] + s2b[slot][c] + eab[slot][c] * wv
                v = jnp.where(v >= 0.0, v, 0.2 * v)
                e = jnp.exp(v)
                exs[slot][c] = e
                exo[slot][c2, pl.ds(j * DIMS, DIMS)] = e
            return carry

        lax.fori_loop(0, C1 // 8, edge_body, 0)

        pltpu.sync_copy(exs[slot], denom_sh.at[didx.at[pl.ds(k * C1, C1)]],
                        add=True)
        pltpu.async_copy(exo[slot], ex_hbm.at[pl.ds(off8, C1 // 8)],
                         xsems[slot])

        @pl.when(k + 2 < CH1)
        def _():
            issue(k + 2, slot)

    issue(0, 0)
    issue(1, 1)

    def pair(j, carry):
        step(2 * j, 0)
        step(2 * j + 1, 1)
        return carry

    lax.fori_loop(0, CH1 // 2, pair, 0)

    pltpu.make_async_copy(
        exo0, ex_hbm.at[pl.ds((base + (CH1 - 2) * C1) // 8, C1 // 8)],
        xsem0).wait()
    pltpu.make_async_copy(
        exo1, ex_hbm.at[pl.ds((base + (CH1 - 1) * C1) // 8, C1 // 8)],
        xsem1).wait()

    plsc.subcore_barrier()
    pltpu.sync_copy(denom_sh.at[pl.ds(sid * ZR, ZR)],
                    dpart_hbm.at[cid, pl.ds(sid * ZR, ZR)])


def _pass1(ei, ea, s1, s2, w16, z16):
    mesh = plsc.VectorSubcoreMesh(core_axis_name="c", subcore_axis_name="s")
    f = functools.partial(
        pl.kernel,
        out_type=[
            jax.ShapeDtypeStruct((E // 8, 8 * DIMS), jnp.float32),
            jax.ShapeDtypeStruct((NC, NPAD, DIMS), jnp.float32),
        ],
        mesh=mesh,
        compiler_params=_SC_PARAMS,
        scratch_types=[
            pltpu.VMEM((EW,), jnp.int32),
            pltpu.VMEM((EW,), jnp.int32),
            pltpu.VMEM((C1, DIMS), jnp.float32),
            pltpu.VMEM((C1, DIMS), jnp.float32),
            pltpu.VMEM((C1, DIMS), jnp.float32),
            pltpu.VMEM((C1, DIMS), jnp.float32),
            pltpu.VMEM((C1, DIMS), jnp.float32),
            pltpu.VMEM((C1, DIMS), jnp.float32),
            pltpu.VMEM((C1 // 8, 8 * DIMS), jnp.float32),
            pltpu.VMEM((C1 // 8, 8 * DIMS), jnp.float32),
            pltpu.VMEM((C1, DIMS), jnp.float32),
            pltpu.VMEM((C1, DIMS), jnp.float32),
            pltpu.VMEM((DIMS,), jnp.float32),
            pltpu.VMEM_SHARED((NPAD, DIMS), jnp.float32),
            pltpu.SemaphoreType.DMA,
            pltpu.SemaphoreType.DMA,
            pltpu.SemaphoreType.DMA,
            pltpu.SemaphoreType.DMA,
        ],
    )(_pass1_body)
    return f(ei, ea, s1, s2, w16, z16)


def _pass2_body(ei_hbm, ex_hbm, inv_hbm, h2_hbm, z128_hbm,
                alpha_hbm, opart_hbm,
                sidx, didx, exb0, exb1, ivb0, ivb1, hb0, hb1, alb0, alb1,
                out_sh, sem0, sem1, asem0, asem1):
    cid = lax.axis_index("c")
    sid = lax.axis_index("s")
    wid = sid * NC + cid
    base = wid * EW

    exb = [exb0, exb1]
    ivb = [ivb0, ivb1]
    hb = [hb0, hb1]
    alb = [alb0, alb1]
    sems = [sem0, sem1]
    asems = [asem0, asem1]

    pltpu.sync_copy(ei_hbm.at[0, pl.ds(base, EW)], sidx)
    pltpu.sync_copy(ei_hbm.at[1, pl.ds(base, EW)], didx)
    pltpu.sync_copy(z128_hbm.at[pl.ds(sid * ZR, ZR)],
                    out_sh.at[pl.ds(sid * ZR, ZR)])
    plsc.subcore_barrier()

    def issue(k, slot):
        off = base + k * C2
        loc = k * C2
        pltpu.async_copy(ex_hbm.at[pl.ds((base + k * C2) // 8, C2 // 8)],
                         exb[slot], sems[slot])
        pltpu.async_copy(inv_hbm.at[didx.at[pl.ds(loc, C2)]], ivb[slot],
                         sems[slot])
        pltpu.async_copy(h2_hbm.at[sidx.at[pl.ds(loc, C2)]], hb[slot],
                         sems[slot])

    def step(k, slot):
        off = base + k * C2

        @pl.when(k >= 2)
        def _():
            pltpu.make_async_copy(
                alb[slot], alpha_hbm.at[pl.ds(base + (k - 2) * C2, C2)],
                asems[slot]).wait()

        pltpu.make_async_copy(ex_hbm.at[pl.ds((base + k * C2) // 8, C2 // 8)],
                              exb[slot], sems[slot]).wait()
        pltpu.make_async_copy(inv_hbm.at[didx.at[pl.ds(k * C2, C2)]],
                              ivb[slot], sems[slot]).wait()
        pltpu.make_async_copy(h2_hbm.at[sidx.at[pl.ds(k * C2, C2)]],
                              hb[slot], sems[slot]).wait()

        def edge_body(c2, carry):
            for j in range(8):
                c = 8 * c2 + j
                a = exb[slot][c2, pl.ds(j * DIMS, DIMS)] * ivb[slot][c]
                alb[slot][c] = a
                for o in range(OUT):
                    hb[slot][c, pl.ds(o * DIMS, DIMS)] = (
                        a * hb[slot][c, pl.ds(o * DIMS, DIMS)])
            return carry

        lax.fori_loop(0, C2 // 8, edge_body, 0)

        pltpu.sync_copy(hb[slot], out_sh.at[didx.at[pl.ds(k * C2, C2)]],
                        add=True)
        pltpu.async_copy(alb[slot], alpha_hbm.at[pl.ds(off, C2)], asems[slot])

        @pl.when(k + 2 < CH2)
        def _():
            issue(k + 2, slot)

    issue(0, 0)
    issue(1, 1)

    def pair(j, carry):
        step(2 * j, 0)
        step(2 * j + 1, 1)
        return carry

    lax.fori_loop(0, CH2 // 2, pair, 0)

    pltpu.make_async_copy(alb0, alpha_hbm.at[pl.ds(base + (CH2 - 2) * C2, C2)],
                          asem0).wait()
    pltpu.make_async_copy(alb1, alpha_hbm.at[pl.ds(base + (CH2 - 1) * C2, C2)],
                          asem1).wait()

    plsc.subcore_barrier()
    pltpu.sync_copy(out_sh.at[pl.ds(sid * ZR, ZR)],
                    opart_hbm.at[cid, pl.ds(sid * ZR, ZR)])


def _pass2(ei, ex, inv, h2, z128):
    mesh = plsc.VectorSubcoreMesh(core_axis_name="c", subcore_axis_name="s")
    f = functools.partial(
        pl.kernel,
        out_type=[
            jax.ShapeDtypeStruct((E, DIMS), jnp.float32),
            jax.ShapeDtypeStruct((NC, NPAD, OUT * DIMS), jnp.float32),
        ],
        mesh=mesh,
        compiler_params=_SC_PARAMS,
        scratch_types=[
            pltpu.VMEM((EW,), jnp.int32),
            pltpu.VMEM((EW,), jnp.int32),
            pltpu.VMEM((C2 // 8, 8 * DIMS), jnp.float32),
            pltpu.VMEM((C2 // 8, 8 * DIMS), jnp.float32),
            pltpu.VMEM((C2, DIMS), jnp.float32),
            pltpu.VMEM((C2, DIMS), jnp.float32),
            pltpu.VMEM((C2, OUT * DIMS), jnp.float32),
            pltpu.VMEM((C2, OUT * DIMS), jnp.float32),
            pltpu.VMEM((C2, DIMS), jnp.float32),
            pltpu.VMEM((C2, DIMS), jnp.float32),
            pltpu.VMEM_SHARED((NPAD, OUT * DIMS), jnp.float32),
            pltpu.SemaphoreType.DMA,
            pltpu.SemaphoreType.DMA,
            pltpu.SemaphoreType.DMA,
            pltpu.SemaphoreType.DMA,
        ],
    )(_pass2_body)
    return f(ei, ex, inv, h2, z128)


def kernel(x, edge_index, edge_attr, W, att):
    r = jnp.arange(OUT * DIMS)
    wp = jnp.transpose(W, (1, 2, 0)).reshape(IN, OUT * DIMS)
    diag = (r[:, None] % DIMS == r[None, :] % DIMS).astype(jnp.float32)
    w2 = jnp.repeat(wp, DIMS, axis=0) * diag
    sel = (r[:, None] % DIMS == jnp.arange(DIMS)[None, :]).astype(jnp.float32)
    a1 = att[:, :OUT].T.reshape(-1)[:, None] * sel
    a2 = att[:, OUT:2 * OUT].T.reshape(-1)[:, None] * sel
    w16 = att[:, 2 * OUT]
    perm = (jnp.arange(OUT * DIMS)[None, :]
            == ((r % DIMS) * OUT + r // DIMS)[:, None]).astype(jnp.float32)
    z16 = jnp.zeros((NPAD, DIMS), jnp.float32)
    z128 = jnp.zeros((NPAD, OUT * DIMS), jnp.float32)

    h2, s1, s2 = _front(x, w2, a1, a2)
    ex, dpart = _pass1(edge_index, edge_attr, s1, s2, w16, z16)
    inv = _mid(dpart)
    alpha, opart = _pass2(edge_index, ex, inv, h2, z128)
    out = _final(opart, perm)
    return out, alpha, edge_index
